# baseline (speedup 1.0000x reference)
import jax
import jax.numpy as jnp
from jax import lax
from jax.experimental import pallas as pl
from jax.experimental.pallas import tpu as pltpu
from jax.experimental.pallas import tpu_sc as plsc

N_FIELDS = 26
VOCAB = 100000
DIM = 32
BATCH = 16384

NC, NS = 2, 16
NW = NC * NS
ROWS = BATCH * N_FIELDS
RPW = ROWS // NW
BLK = 128
NBLK = RPW // BLK


def _body(idx_hbm, tab_hbm, out_hbm, idx_v, rows_v, sem):
    wid = lax.axis_index("s") * NC + lax.axis_index("c")
    pltpu.sync_copy(idx_hbm.at[pl.ds(wid * NBLK, NBLK), :], idx_v)

    @pl.loop(0, NBLK)
    def _(j):
        pltpu.async_copy(tab_hbm.at[idx_v.at[j]], rows_v, sem).wait()
        pltpu.sync_copy(rows_v, out_hbm.at[pl.ds(wid * RPW + j * BLK, BLK), :])


_gather = pl.kernel(
    _body,
    out_type=jax.ShapeDtypeStruct((ROWS, DIM), jnp.float32),
    mesh=plsc.VectorSubcoreMesh(core_axis_name="c", subcore_axis_name="s"),
    scratch_types=[
        pltpu.VMEM((NBLK, BLK), jnp.int32),
        pltpu.VMEM((BLK, DIM), jnp.float32),
        pltpu.SemaphoreType.DMA,
    ],
    compiler_params=pltpu.CompilerParams(use_tc_tiling_on_sc=False),
)


def kernel(one_hot_x, tables):
    offs = (jnp.arange(N_FIELDS, dtype=jnp.int32) * (VOCAB + 1))[None, :]
    flat_idx = (one_hot_x + offs).reshape(ROWS // BLK, BLK)
    tab = tables.reshape(N_FIELDS * (VOCAB + 1), DIM)
    out = _gather(flat_idx, tab)
    return out.reshape(BATCH, N_FIELDS, DIM)


# speedup vs baseline: 1.0050x; 1.0050x over previous
import jax
import jax.numpy as jnp
from jax import lax
from jax.experimental import pallas as pl
from jax.experimental.pallas import tpu as pltpu
from jax.experimental.pallas import tpu_sc as plsc

N_FIELDS = 26
VOCAB = 100000
DIM = 32
BATCH = 16384

NC, NS = 2, 16
NW = NC * NS
ROWS = BATCH * N_FIELDS
RPW = ROWS // NW
BLK = 128
NBLK = RPW // BLK


NBUF = 4
WAVES = NBLK // NBUF


def _body(idx_hbm, tab_hbm, out_hbm, idx_v,
          a0, a1, a2, a3, b0, b1, b2, b3, sga, sgb, soa, sob):
    bufs_a = (a0, a1, a2, a3)
    bufs_b = (b0, b1, b2, b3)
    wid = lax.axis_index("s") * NC + lax.axis_index("c")
    pltpu.sync_copy(idx_hbm.at[pl.ds(wid * NBLK, NBLK), :], idx_v)
    base = wid * RPW

    def gathers(w, bufs, sem):
        for b in range(NBUF):
            pltpu.async_copy(tab_hbm.at[idx_v.at[w * NBUF + b]], bufs[b], sem)

    def outs(w, bufs, sem):
        for b in range(NBUF):
            pltpu.async_copy(
                bufs[b], out_hbm.at[pl.ds(base + (w * NBUF + b) * BLK, BLK), :], sem)

    def drain(sem):
        for _ in range(NBUF):
            pltpu.make_async_copy(tab_hbm.at[pl.ds(0, BLK), :], bufs_a[0], sem).wait()

    gathers(0, bufs_a, sga)

    @pl.loop(0, WAVES // 2)
    def _(p):
        w = p * 2
        drain(sga)

        @pl.when(p > 0)
        def _():
            drain(sob)

        gathers(w + 1, bufs_b, sgb)
        outs(w, bufs_a, soa)

        drain(sgb)
        drain(soa)

        @pl.when(w + 2 < WAVES)
        def _():
            gathers(w + 2, bufs_a, sga)

        outs(w + 1, bufs_b, sob)

    drain(sob)


_gather = pl.kernel(
    _body,
    out_type=jax.ShapeDtypeStruct((ROWS, DIM), jnp.float32),
    mesh=plsc.VectorSubcoreMesh(core_axis_name="c", subcore_axis_name="s"),
    scratch_types=(
        [pltpu.VMEM((NBLK, BLK), jnp.int32)]
        + [pltpu.VMEM((BLK, DIM), jnp.float32) for _ in range(2 * NBUF)]
        + [pltpu.SemaphoreType.DMA for _ in range(4)]
    ),
    compiler_params=pltpu.CompilerParams(use_tc_tiling_on_sc=False),
)


def kernel(one_hot_x, tables):
    offs = (jnp.arange(N_FIELDS, dtype=jnp.int32) * (VOCAB + 1))[None, :]
    flat_idx = (one_hot_x + offs).reshape(ROWS // BLK, BLK)
    tab = tables.reshape(N_FIELDS * (VOCAB + 1), DIM)
    out = _gather(flat_idx, tab)
    return out.reshape(BATCH, N_FIELDS, DIM)


# speedup vs baseline: 30.1036x; 29.9530x over previous
import jax
import jax.numpy as jnp
from jax import lax
from jax.experimental import pallas as pl
from jax.experimental.pallas import tpu as pltpu
from jax.experimental.pallas import tpu_sc as plsc

N_FIELDS = 26
VOCAB = 100000
DIM = 32
BATCH = 16384

NC, NS, L = 2, 16, 16
NW = NC * NS
TASKS = N_FIELDS * DIM
TPW = TASKS // NW
OUTCH = 2048


def _body(idx_hbm, tab_hbm, out_hbm, idx_v, row_v, out_v):
    wid = lax.axis_index("s") * NC + lax.axis_index("c")

    @pl.loop(0, TPW)
    def _(k):
        task = wid * TPW + k
        f = task // DIM
        d = task % DIM

        @pl.when(jnp.logical_or(k == 0, d == 0))
        def _():
            pltpu.sync_copy(idx_hbm.at[f], idx_v)

        pltpu.sync_copy(tab_hbm.at[f, d], row_v)

        @pl.loop(0, BATCH // OUTCH)
        def _(c):
            @pl.loop(0, OUTCH // L)
            def _(i):
                v = idx_v[pl.ds(c * OUTCH + i * L, L)]
                out_v[pl.ds(i * L, L)] = plsc.load_gather(row_v, [v])

            pltpu.sync_copy(out_v, out_hbm.at[f, d, pl.ds(c * OUTCH, OUTCH)])


_gather = pl.kernel(
    _body,
    out_type=jax.ShapeDtypeStruct((N_FIELDS, DIM, BATCH), jnp.float32),
    mesh=plsc.VectorSubcoreMesh(core_axis_name="c", subcore_axis_name="s"),
    scratch_types=[
        pltpu.VMEM((BATCH,), jnp.int32),
        pltpu.VMEM((VOCAB + 1,), jnp.float32),
        pltpu.VMEM((OUTCH,), jnp.float32),
    ],
    compiler_params=pltpu.CompilerParams(
        use_tc_tiling_on_sc=True, needs_layout_passes=False),
)


def kernel(one_hot_x, tables):
    idx_t = jnp.transpose(one_hot_x, (1, 0))
    tab_t = jnp.transpose(tables, (0, 2, 1))
    out = _gather(idx_t, tab_t)
    return jnp.transpose(out, (2, 0, 1))
